# Initial kernel scaffold; baseline (speedup 1.0000x reference)
import functools

import jax
import jax.numpy as jnp
from jax import lax
from jax.experimental import pallas as pl
from jax.experimental.pallas import tpu as pltpu
from jax.experimental.pallas import tpu_sc as plsc

N = 10000
D = 128
DA = 144
E = 320000
NC = 2
NS = 16
NW = NC * NS
EPW = E // NW
CHUNK = 80
NCHUNK = EPW // CHUNK
RPS = N // NS


def _sc_aggregate(h_aug, src, dst, zeros_blk):
    mesh = plsc.VectorSubcoreMesh(core_axis_name="c", subcore_axis_name="s")

    @functools.partial(
        pl.kernel,
        out_type=jax.ShapeDtypeStruct((NC, N, DA), jnp.float32),
        mesh=mesh,
        scratch_types=[
            pltpu.VMEM((NCHUNK, CHUNK), jnp.int32),
            pltpu.VMEM((NCHUNK, CHUNK), jnp.int32),
            pltpu.VMEM((CHUNK, DA), jnp.float32),
            pltpu.VMEM_SHARED((N, DA), jnp.float32),
            pltpu.SemaphoreType.DMA,
        ],
    )
    def agg(h_hbm, src_hbm, dst_hbm, z_hbm, out_hbm,
            src_v, dst_v, rows_v, acc_sh, sem):
        c = lax.axis_index("c")
        s = lax.axis_index("s")
        wid = c * NS + s

        pltpu.sync_copy(z_hbm, acc_sh.at[pl.ds(s * RPS, RPS)])
        pltpu.sync_copy(src_hbm.at[wid], src_v)
        pltpu.sync_copy(dst_hbm.at[wid], dst_v)
        plsc.subcore_barrier()

        @pl.loop(0, NCHUNK)
        def _(j):
            pltpu.async_copy(h_hbm.at[src_v.at[j]], rows_v, sem).wait()
            pltpu.sync_copy(rows_v, acc_sh.at[dst_v.at[j]], add=True)

        plsc.subcore_barrier()
        pltpu.sync_copy(acc_sh.at[pl.ds(s * RPS, RPS)],
                        out_hbm.at[c, pl.ds(s * RPS, RPS)])

    return agg(h_aug, src, dst, zeros_blk)


def _layer_body(h_ref, acc_ref, ws_ref, wn_ref, b_ref, out_ref, *, relu, aug):
    h = h_ref[:, 0:D]
    a = acc_ref[0] + acc_ref[1]
    deg = jnp.maximum(a[:, D:D + 1], 1.0)
    hn = a[:, 0:D] / deg
    dn = (((1,), (1,)), ((), ()))
    out = lax.dot_general(h, ws_ref[...], dn,
                          preferred_element_type=jnp.float32,
                          precision=lax.Precision.HIGHEST)
    out = out + lax.dot_general(hn, wn_ref[...], dn,
                                preferred_element_type=jnp.float32,
                                precision=lax.Precision.HIGHEST)
    out = out + b_ref[...]
    if relu:
        out = jnp.maximum(out, 0.0)
    if aug:
        col = lax.broadcasted_iota(jnp.int32, (out.shape[0], DA - D), 1)
        pad = jnp.where(col == 0, 1.0, 0.0).astype(jnp.float32)
        out = jnp.concatenate([out, pad], axis=1)
    out_ref[...] = out


def _tc_layer(h, acc, W_self, W_neigh, b, *, relu, aug):
    BS = 1000
    hw = h.shape[1]
    ow = DA if aug else D
    return pl.pallas_call(
        functools.partial(_layer_body, relu=relu, aug=aug),
        grid=(N // BS,),
        in_specs=[
            pl.BlockSpec((BS, hw), lambda i: (i, 0)),
            pl.BlockSpec((NC, BS, DA), lambda i: (0, i, 0)),
            pl.BlockSpec((D, D), lambda i: (0, 0)),
            pl.BlockSpec((D, D), lambda i: (0, 0)),
            pl.BlockSpec((1, D), lambda i: (0, 0)),
        ],
        out_specs=pl.BlockSpec((BS, ow), lambda i: (i, 0)),
        out_shape=jax.ShapeDtypeStruct((N, ow), jnp.float32),
    )(h, acc, W_self, W_neigh, b)


def kernel(feat, edge_index, W_self0, W_neigh0, b0, W_self1, W_neigh1, b1):
    src = edge_index[0].astype(jnp.int32).reshape(NW, NCHUNK, CHUNK)
    dst = edge_index[1].astype(jnp.int32).reshape(NW, NCHUNK, CHUNK)
    feat_aug = jnp.concatenate(
        [feat, jnp.ones((N, 1), jnp.float32), jnp.zeros((N, DA - D - 1), jnp.float32)],
        axis=1)
    zeros_blk = jnp.zeros((RPS, DA), jnp.float32)
    b0r = b0.reshape(1, D)
    b1r = b1.reshape(1, D)

    acc0 = _sc_aggregate(feat_aug, src, dst, zeros_blk)
    h1 = _tc_layer(feat, acc0, W_self0, W_neigh0, b0r, relu=True, aug=True)
    acc1 = _sc_aggregate(h1, src, dst, zeros_blk)
    out = _tc_layer(h1, acc1, W_self1, W_neigh1, b1r, relu=False, aug=False)
    return out


# speedup vs baseline: 6.5130x; 6.5130x over previous
import functools

import jax
import jax.numpy as jnp
from jax import lax
from jax.experimental import pallas as pl
from jax.experimental.pallas import tpu as pltpu
from jax.experimental.pallas import tpu_sc as plsc

N = 10000
D = 128
DA = 144
E = 320000
NC = 2
NS = 16
NW = NC * NS
EPW = E // NW
CHUNK = 80
NCHUNK = EPW // CHUNK
RPS = 632
NP = NS * RPS


def _sc_aggregate(h_aug, src, dst, zeros_blk):
    mesh = plsc.VectorSubcoreMesh(core_axis_name="c", subcore_axis_name="s")

    @functools.partial(
        pl.kernel,
        out_type=jax.ShapeDtypeStruct((NC, NP, DA), jnp.float32),
        mesh=mesh,
        scratch_types=[
            pltpu.VMEM((NCHUNK, CHUNK), jnp.int32),
            pltpu.VMEM((NCHUNK, CHUNK), jnp.int32),
            pltpu.VMEM((CHUNK, DA), jnp.float32),
            pltpu.VMEM_SHARED((NP, DA), jnp.float32),
            pltpu.SemaphoreType.DMA,
        ],
        compiler_params=pltpu.CompilerParams(use_tc_tiling_on_sc=False),
    )
    def agg(h_hbm, src_hbm, dst_hbm, z_hbm, out_hbm,
            src_v, dst_v, rows_v, acc_sh, sem):
        c = lax.axis_index("c")
        s = lax.axis_index("s")
        wid = c * NS + s

        pltpu.sync_copy(z_hbm, acc_sh.at[pl.ds(s * RPS, RPS)])
        pltpu.sync_copy(src_hbm.at[wid], src_v)
        pltpu.sync_copy(dst_hbm.at[wid], dst_v)
        plsc.subcore_barrier()

        @pl.loop(0, NCHUNK)
        def _(j):
            pltpu.async_copy(h_hbm.at[src_v.at[j]], rows_v, sem).wait()
            pltpu.sync_copy(rows_v, acc_sh.at[dst_v.at[j]], add=True)

        plsc.subcore_barrier()
        pltpu.sync_copy(acc_sh.at[pl.ds(s * RPS, RPS)],
                        out_hbm.at[c, pl.ds(s * RPS, RPS)])

    return agg(h_aug, src, dst, zeros_blk)


def _layer_body(h_ref, acc_ref, ws_ref, wn_ref, b_ref, out_ref, *, relu, aug):
    h = h_ref[:, 0:D]
    a = acc_ref[0] + acc_ref[1]
    deg = jnp.maximum(a[:, D:D + 1], 1.0)
    hn = a[:, 0:D] / deg
    dn = (((1,), (1,)), ((), ()))
    out = lax.dot_general(h, ws_ref[...], dn,
                          preferred_element_type=jnp.float32,
                          precision=lax.Precision.HIGHEST)
    out = out + lax.dot_general(hn, wn_ref[...], dn,
                                preferred_element_type=jnp.float32,
                                precision=lax.Precision.HIGHEST)
    out = out + b_ref[...]
    if relu:
        out = jnp.maximum(out, 0.0)
    if aug:
        col = lax.broadcasted_iota(jnp.int32, (out.shape[0], DA - D), 1)
        pad = jnp.where(col == 0, 1.0, 0.0).astype(jnp.float32)
        out = jnp.concatenate([out, pad], axis=1)
    out_ref[...] = out


def _tc_layer(h, acc, W_self, W_neigh, b, *, relu, aug):
    BS = 1000
    hw = h.shape[1]
    ow = DA if aug else D
    return pl.pallas_call(
        functools.partial(_layer_body, relu=relu, aug=aug),
        grid=(N // BS,),
        in_specs=[
            pl.BlockSpec((BS, hw), lambda i: (i, 0)),
            pl.BlockSpec((NC, BS, DA), lambda i: (0, i, 0)),
            pl.BlockSpec((D, D), lambda i: (0, 0)),
            pl.BlockSpec((D, D), lambda i: (0, 0)),
            pl.BlockSpec((1, D), lambda i: (0, 0)),
        ],
        out_specs=pl.BlockSpec((BS, ow), lambda i: (i, 0)),
        out_shape=jax.ShapeDtypeStruct((N, ow), jnp.float32),
    )(h, acc, W_self, W_neigh, b)


def kernel(feat, edge_index, W_self0, W_neigh0, b0, W_self1, W_neigh1, b1):
    src = edge_index[0].astype(jnp.int32).reshape(NW, NCHUNK, CHUNK)
    dst = edge_index[1].astype(jnp.int32).reshape(NW, NCHUNK, CHUNK)
    feat_aug = jnp.concatenate(
        [feat, jnp.ones((N, 1), jnp.float32), jnp.zeros((N, DA - D - 1), jnp.float32)],
        axis=1)
    zeros_blk = jnp.zeros((RPS, DA), jnp.float32)
    b0r = b0.reshape(1, D)
    b1r = b1.reshape(1, D)

    acc0 = _sc_aggregate(feat_aug, src, dst, zeros_blk)
    h1 = _tc_layer(feat, acc0, W_self0, W_neigh0, b0r, relu=True, aug=True)
    acc1 = _sc_aggregate(h1, src, dst, zeros_blk)
    out = _tc_layer(h1, acc1, W_self1, W_neigh1, b1r, relu=False, aug=False)
    return out
